# baseline (speedup 1.0000x reference)
import functools

import jax
import jax.numpy as jnp
from jax import lax
from jax.experimental import pallas as pl
from jax.experimental.pallas import tpu as pltpu
from jax.experimental.pallas import tpu_sc as plsc

NS = 16
L = 16


@functools.lru_cache(maxsize=None)
def _build_hist(n_idx):
    ic = n_idx // NS
    mesh = plsc.VectorSubcoreMesh(
        core_axis_name="c", subcore_axis_name="s", num_cores=1, num_subcores=NS
    )

    @functools.partial(
        pl.kernel,
        out_type=jax.ShapeDtypeStruct((NS, 64), jnp.float32),
        mesh=mesh,
        scratch_types=[
            pltpu.VMEM((ic,), jnp.int32),
            pltpu.VMEM((128,), jnp.float32),
            pltpu.VMEM((64,), jnp.float32),
        ],
        compiler_params=pltpu.CompilerParams(needs_layout_passes=False),
    )
    def hist(idx_hbm, out_hbm, idx_v, part_v, cnt_v):
        sid = lax.axis_index("s")
        pltpu.sync_copy(idx_hbm.at[pl.ds(sid * ic, ic)], idx_v)

        zeros = jnp.zeros((L,), jnp.float32)
        ones = jnp.ones((L,), jnp.float32)
        for j in range(8):
            part_v[pl.ds(j * L, L)] = zeros

        off = (lax.iota(jnp.int32, 16) & 1) * 64
        for k in range(ic // L):
            idx = idx_v[pl.ds(k * L, L)]
            plsc.addupdate_scatter(part_v, [idx + off], ones)

        for j in range(4):
            cnt_v[pl.ds(j * L, L)] = (
                part_v[pl.ds(j * L, L)] + part_v[pl.ds(64 + j * L, L)]
            )
        pltpu.sync_copy(cnt_v, out_hbm.at[sid])

    return hist


@functools.lru_cache(maxsize=None)
def _build_colsum_dot(rows, cols, scale, tiles=1):
    tile = rows // tiles

    def body(parts_ref, x_ref, o_ref, acc_ref):
        i = pl.program_id(0)

        @pl.when(i == 0)
        def _():
            acc_ref[...] = jnp.zeros_like(acc_ref)

        acc_ref[...] += x_ref[...]

        @pl.when(i == tiles - 1)
        def _():
            cnt = jnp.transpose(jnp.sum(parts_ref[...], axis=0, keepdims=True))
            o_ref[0] = jnp.sum(acc_ref[...] * cnt) * scale

    return pl.pallas_call(
        body,
        grid=(tiles,),
        in_specs=[
            pl.BlockSpec((NS, cols), lambda i: (0, 0)),
            pl.BlockSpec((cols, tile), lambda i: (0, i)),
        ],
        out_specs=pl.BlockSpec(memory_space=pltpu.SMEM),
        out_shape=jax.ShapeDtypeStruct((1,), jnp.float32),
        scratch_shapes=[pltpu.VMEM((cols, tile), jnp.float32)],
    )


def kernel(router_probs, expert_indices, num_experts):
    B, M = router_probs.shape
    K = expert_indices.shape[1]
    assert M == 64, "kernel specialized for 64 experts"
    del num_experts
    idx_flat = expert_indices.T.reshape(-1).astype(jnp.int32)
    parts = _build_hist(B * K)(idx_flat)
    scale = float(M) / (float(B) * K * B)
    out = _build_colsum_dot(B, M, scale)(parts, router_probs.T)
    return out[0]


# speedup vs baseline: 1.0039x; 1.0039x over previous
import functools

import jax
import jax.numpy as jnp
from jax import lax
from jax.experimental import pallas as pl
from jax.experimental.pallas import tpu as pltpu
from jax.experimental.pallas import tpu_sc as plsc

NS = 16
L = 16


@functools.lru_cache(maxsize=None)
def _build_hist(n_rows, n_cols):
    ic = n_rows * n_cols // NS
    per_row = NS // n_rows
    mesh = plsc.VectorSubcoreMesh(
        core_axis_name="c", subcore_axis_name="s", num_cores=1, num_subcores=NS
    )

    @functools.partial(
        pl.kernel,
        out_type=jax.ShapeDtypeStruct((NS, 64), jnp.float32),
        mesh=mesh,
        scratch_types=[
            pltpu.VMEM((ic,), jnp.int32),
            pltpu.VMEM((128,), jnp.float32),
            pltpu.VMEM((64,), jnp.float32),
        ],
        compiler_params=pltpu.CompilerParams(needs_layout_passes=False),
    )
    def hist(idx_hbm, out_hbm, idx_v, part_v, cnt_v):
        sid = lax.axis_index("s")
        pltpu.sync_copy(
            idx_hbm.at[sid // per_row, pl.ds((sid % per_row) * ic, ic)], idx_v
        )

        zeros = jnp.zeros((L,), jnp.float32)
        ones = jnp.ones((L,), jnp.float32)
        for j in range(8):
            part_v[pl.ds(j * L, L)] = zeros

        off = (lax.iota(jnp.int32, 16) & 1) * 64
        for k in range(ic // L):
            idx = idx_v[pl.ds(k * L, L)]
            plsc.addupdate_scatter(part_v, [idx + off], ones)

        for j in range(4):
            cnt_v[pl.ds(j * L, L)] = (
                part_v[pl.ds(j * L, L)] + part_v[pl.ds(64 + j * L, L)]
            )
        pltpu.sync_copy(cnt_v, out_hbm.at[sid])

    return hist


@functools.lru_cache(maxsize=None)
def _build_colsum_dot(rows, cols, scale, tiles=2):
    tile = rows // tiles

    def body(parts_ref, x_ref, o_ref, acc_ref):
        i = pl.program_id(0)

        @pl.when(i == 0)
        def _():
            acc_ref[...] = jnp.zeros_like(acc_ref)

        acc_ref[...] += x_ref[...]

        @pl.when(i == tiles - 1)
        def _():
            cnt = jnp.transpose(jnp.sum(parts_ref[...], axis=0, keepdims=True))
            o_ref[0] = jnp.sum(acc_ref[...] * cnt) * scale

    return pl.pallas_call(
        body,
        grid=(tiles,),
        in_specs=[
            pl.BlockSpec((NS, cols), lambda i: (0, 0)),
            pl.BlockSpec((cols, tile), lambda i: (0, i)),
        ],
        out_specs=pl.BlockSpec(memory_space=pltpu.SMEM),
        out_shape=jax.ShapeDtypeStruct((1,), jnp.float32),
        scratch_shapes=[pltpu.VMEM((cols, tile), jnp.float32)],
    )


def kernel(router_probs, expert_indices, num_experts):
    B, M = router_probs.shape
    K = expert_indices.shape[1]
    assert M == 64, "kernel specialized for 64 experts"
    del num_experts
    idx_t = expert_indices.T.astype(jnp.int32)
    parts = _build_hist(K, B)(idx_t)
    scale = float(M) / (float(B) * K * B)
    out = _build_colsum_dot(B, M, scale)(parts, router_probs.T)
    return out[0]


# speedup vs baseline: 1.0134x; 1.0095x over previous
import functools

import jax
import jax.numpy as jnp
from jax import lax
from jax.experimental import pallas as pl
from jax.experimental.pallas import tpu as pltpu
from jax.experimental.pallas import tpu_sc as plsc

NS = 16
L = 16


@functools.lru_cache(maxsize=None)
def _build_hist(n_idx):
    ic = n_idx // NS
    mesh = plsc.VectorSubcoreMesh(
        core_axis_name="c", subcore_axis_name="s", num_cores=1, num_subcores=NS
    )

    @functools.partial(
        pl.kernel,
        out_type=jax.ShapeDtypeStruct((NS, 64), jnp.float32),
        mesh=mesh,
        scratch_types=[
            pltpu.VMEM((ic,), jnp.int32),
            pltpu.VMEM((128,), jnp.float32),
            pltpu.VMEM((64,), jnp.float32),
        ],
        compiler_params=pltpu.CompilerParams(needs_layout_passes=False),
    )
    def hist(idx_hbm, out_hbm, idx_v, part_v, cnt_v):
        sid = lax.axis_index("s")
        pltpu.sync_copy(idx_hbm.at[pl.ds(sid * ic, ic)], idx_v)

        zeros = jnp.zeros((L,), jnp.float32)
        ones = jnp.ones((L,), jnp.float32)
        for j in range(8):
            part_v[pl.ds(j * L, L)] = zeros

        off = (lax.iota(jnp.int32, 16) & 1) * 64
        for k in range(ic // L):
            idx = idx_v[pl.ds(k * L, L)]
            plsc.addupdate_scatter(part_v, [idx + off], ones)

        for j in range(4):
            cnt_v[pl.ds(j * L, L)] = (
                part_v[pl.ds(j * L, L)] + part_v[pl.ds(64 + j * L, L)]
            )
        pltpu.sync_copy(cnt_v, out_hbm.at[sid])

    return hist


@functools.lru_cache(maxsize=None)
def _build_colsum_dot(rows, cols, scale, tiles=2):
    tile = rows // tiles

    def body(parts_ref, x_ref, o_ref, acc_ref):
        i = pl.program_id(0)

        @pl.when(i == 0)
        def _():
            acc_ref[...] = jnp.zeros_like(acc_ref)

        acc_ref[...] += x_ref[...]

        @pl.when(i == tiles - 1)
        def _():
            cnt = jnp.transpose(jnp.sum(parts_ref[...], axis=0, keepdims=True))
            o_ref[0] = jnp.sum(acc_ref[...] * cnt) * scale

    return pl.pallas_call(
        body,
        grid=(tiles,),
        in_specs=[
            pl.BlockSpec((NS, cols), lambda i: (0, 0)),
            pl.BlockSpec((cols, tile), lambda i: (0, i)),
        ],
        out_specs=pl.BlockSpec(memory_space=pltpu.SMEM),
        out_shape=jax.ShapeDtypeStruct((1,), jnp.float32),
        scratch_shapes=[pltpu.VMEM((cols, tile), jnp.float32)],
    )


def kernel(router_probs, expert_indices, num_experts):
    B, M = router_probs.shape
    K = expert_indices.shape[1]
    assert M == 64, "kernel specialized for 64 experts"
    del num_experts
    idx_flat = expert_indices.T.reshape(-1).astype(jnp.int32)
    parts = _build_hist(B * K)(idx_flat)
    scale = float(M) / (float(B) * K * B)
    out = _build_colsum_dot(B, M, scale)(parts, router_probs.T)
    return out[0]
